# baseline (speedup 1.0000x reference)
import jax
import jax.numpy as jnp
from jax.experimental import pallas as pl
from jax.experimental.pallas import tpu as pltpu

C_IN, D_OUT, N_ETAS, H_GRID, W_GRID = 512, 512, 8, 64, 128
N_COLS = H_GRID * W_GRID
H_BLK = 8
GRID = H_GRID // H_BLK


def _fused_body(eta_ref, x_ref, bW_ref, bb_ref, eW_ref, eb_ref, out_ref, bp_ref):
    bW = bW_ref[...].astype(jnp.bfloat16)
    for h in range(H_BLK):
        xh = x_ref[:, h, :].astype(jnp.bfloat16)
        bp = jax.lax.dot_general(xh, bW, (((0,), (0,)), ((), ())),
                                 preferred_element_type=jnp.float32)
        bp_ref[pl.ds(h * W_GRID, W_GRID), :] = bp + bb_ref[...]
        eta_h = eta_ref[h]
        acc = jnp.zeros((D_OUT, W_GRID), jnp.float32)
        for e in range(N_ETAS):
            oe = jax.lax.dot_general(eW_ref[e].astype(jnp.bfloat16), xh,
                                     (((0,), (0,)), ((), ())),
                                     preferred_element_type=jnp.float32)
            acc = jnp.where(eta_h == e, oe + eb_ref[e], acc)
        out_ref[:, h, :] = acc


def kernel(x, eta, base_W, base_b, expert_W, expert_b):
    bb2 = base_b.reshape(1, D_OUT)
    eb3 = expert_b.reshape(N_ETAS, D_OUT, 1)

    out3, bp = pl.pallas_call(
        _fused_body,
        grid=(GRID,),
        in_specs=[
            pl.BlockSpec((H_BLK, W_GRID), lambda i: (i, 0)),
            pl.BlockSpec((C_IN, H_BLK, W_GRID), lambda i: (0, i, 0)),
            pl.BlockSpec((C_IN, D_OUT), lambda i: (0, 0)),
            pl.BlockSpec((1, D_OUT), lambda i: (0, 0)),
            pl.BlockSpec((N_ETAS, C_IN, D_OUT), lambda i: (0, 0, 0)),
            pl.BlockSpec((N_ETAS, D_OUT, 1), lambda i: (0, 0, 0)),
        ],
        out_specs=[
            pl.BlockSpec((D_OUT, H_BLK, W_GRID), lambda i: (0, i, 0)),
            pl.BlockSpec((H_BLK * W_GRID, D_OUT), lambda i: (i, 0)),
        ],
        out_shape=[
            jax.ShapeDtypeStruct((D_OUT, H_GRID, W_GRID), jnp.float32),
            jax.ShapeDtypeStruct((N_COLS, D_OUT), jnp.float32),
        ],
        compiler_params=pltpu.CompilerParams(
            dimension_semantics=("parallel",)),
    )(eta, x, base_W, bb2, expert_W, eb3)

    return out3, bp


# speedup vs baseline: 2.2501x; 2.2501x over previous
import jax
import jax.numpy as jnp
from jax.experimental import pallas as pl
from jax.experimental.pallas import tpu as pltpu

C_IN, D_OUT, N_ETAS, H_GRID, W_GRID = 512, 512, 8, 64, 128
N_COLS = H_GRID * W_GRID
H_BLK = 8
T_N = H_BLK * W_GRID
GRID = H_GRID // H_BLK


def _fused_body(eta_ref, x_ref, bW_ref, bb_ref, eW_ref, eb_ref, out_ref, bp_ref):
    xcat = jnp.concatenate(
        [x_ref[:, h, :] for h in range(H_BLK)], axis=1).astype(jnp.bfloat16)
    etacat = jnp.concatenate([eta_ref[h] for h in range(H_BLK)], axis=0)
    bp = jax.lax.dot_general(xcat, bW_ref[...].astype(jnp.bfloat16),
                             (((0,), (0,)), ((), ())),
                             preferred_element_type=jnp.float32)
    bp_ref[...] = bp + bb_ref[...]
    acc = jnp.zeros((D_OUT, T_N), jnp.float32)
    for e in range(N_ETAS):
        oe = jax.lax.dot_general(eW_ref[e].astype(jnp.bfloat16), xcat,
                                 (((0,), (0,)), ((), ())),
                                 preferred_element_type=jnp.float32)
        acc = jnp.where(etacat == e, oe + eb_ref[e], acc)
    for h in range(H_BLK):
        out_ref[:, h, :] = acc[:, h * W_GRID:(h + 1) * W_GRID]


def kernel(x, eta, base_W, base_b, expert_W, expert_b):
    bb2 = base_b.reshape(1, D_OUT)
    eb3 = expert_b.reshape(N_ETAS, D_OUT, 1)

    out3, bp = pl.pallas_call(
        _fused_body,
        grid=(GRID,),
        in_specs=[
            pl.BlockSpec((H_BLK, W_GRID), lambda i: (i, 0)),
            pl.BlockSpec((C_IN, H_BLK, W_GRID), lambda i: (0, i, 0)),
            pl.BlockSpec((C_IN, D_OUT), lambda i: (0, 0)),
            pl.BlockSpec((1, D_OUT), lambda i: (0, 0)),
            pl.BlockSpec((N_ETAS, C_IN, D_OUT), lambda i: (0, 0, 0)),
            pl.BlockSpec((N_ETAS, D_OUT, 1), lambda i: (0, 0, 0)),
        ],
        out_specs=[
            pl.BlockSpec((D_OUT, H_BLK, W_GRID), lambda i: (0, i, 0)),
            pl.BlockSpec((T_N, D_OUT), lambda i: (i, 0)),
        ],
        out_shape=[
            jax.ShapeDtypeStruct((D_OUT, H_GRID, W_GRID), jnp.float32),
            jax.ShapeDtypeStruct((N_COLS, D_OUT), jnp.float32),
        ],
        compiler_params=pltpu.CompilerParams(
            dimension_semantics=("parallel",)),
    )(eta, x, base_W, bb2, expert_W, eb3)

    return out3, bp


# speedup vs baseline: 2.2974x; 1.0210x over previous
import jax
import jax.numpy as jnp
from jax.experimental import pallas as pl
from jax.experimental.pallas import tpu as pltpu

C_IN, D_OUT, N_ETAS, H_GRID, W_GRID = 512, 512, 8, 64, 128
N_COLS = H_GRID * W_GRID
H_BLK = 16
T_N = H_BLK * W_GRID
GRID = H_GRID // H_BLK


def _fused_body(eta_ref, x_ref, bW_ref, bb_ref, eW_ref, eb_ref, out_ref, bp_ref):
    xcat = jnp.concatenate(
        [x_ref[:, h, :] for h in range(H_BLK)], axis=1).astype(jnp.bfloat16)
    etacat = jnp.concatenate([eta_ref[h] for h in range(H_BLK)], axis=0)
    bp = jax.lax.dot_general(xcat, bW_ref[...].astype(jnp.bfloat16),
                             (((0,), (0,)), ((), ())),
                             preferred_element_type=jnp.float32)
    bp_ref[...] = bp + bb_ref[...]
    acc = jnp.zeros((D_OUT, T_N), jnp.float32)
    for e in range(N_ETAS):
        oe = jax.lax.dot_general(eW_ref[e].astype(jnp.bfloat16), xcat,
                                 (((0,), (0,)), ((), ())),
                                 preferred_element_type=jnp.float32)
        acc = jnp.where(etacat == e, oe + eb_ref[e], acc)
    for h in range(H_BLK):
        out_ref[:, h, :] = acc[:, h * W_GRID:(h + 1) * W_GRID]


def kernel(x, eta, base_W, base_b, expert_W, expert_b):
    bb2 = base_b.reshape(1, D_OUT)
    eb3 = expert_b.reshape(N_ETAS, D_OUT, 1)

    out3, bp = pl.pallas_call(
        _fused_body,
        grid=(GRID,),
        in_specs=[
            pl.BlockSpec((H_BLK, W_GRID), lambda i: (i, 0)),
            pl.BlockSpec((C_IN, H_BLK, W_GRID), lambda i: (0, i, 0)),
            pl.BlockSpec((C_IN, D_OUT), lambda i: (0, 0)),
            pl.BlockSpec((1, D_OUT), lambda i: (0, 0)),
            pl.BlockSpec((N_ETAS, C_IN, D_OUT), lambda i: (0, 0, 0)),
            pl.BlockSpec((N_ETAS, D_OUT, 1), lambda i: (0, 0, 0)),
        ],
        out_specs=[
            pl.BlockSpec((D_OUT, H_BLK, W_GRID), lambda i: (0, i, 0)),
            pl.BlockSpec((T_N, D_OUT), lambda i: (i, 0)),
        ],
        out_shape=[
            jax.ShapeDtypeStruct((D_OUT, H_GRID, W_GRID), jnp.float32),
            jax.ShapeDtypeStruct((N_COLS, D_OUT), jnp.float32),
        ],
        compiler_params=pltpu.CompilerParams(
            dimension_semantics=("parallel",)),
    )(eta, x, base_W, bb2, expert_W, eb3)

    return out3, bp
